# baseline (speedup 1.0000x reference)
import functools
import jax
import jax.numpy as jnp
from jax import lax
from jax.experimental import pallas as pl
from jax.experimental.pallas import tpu as pltpu
from jax.experimental.pallas import tpu_sc as plsc

N_U = 10000
N_I = 10000
N = N_U + N_I
D = 128
E = 160000
H = 4
DH = 32
TEMP = 0.2
LAMBDA_1 = 0.2
LAMBDA_2 = 1e-07
NEG_SLOPE = 0.2
B = 1024

NS = 16
CH = 128

SP_PER_TILE = 10240
SP_CHUNKS = SP_PER_TILE // CH
SP_LEN = NS * SP_PER_TILE

GCH = 96
GA_PER_TILE = 10752
GA_CHUNKS = GA_PER_TILE // GCH
GA_LEN = NS * GA_PER_TILE
NLOC = 10016
STRIPE = NLOC // NS


def _mesh():
  return plsc.VectorSubcoreMesh(core_axis_name="c", subcore_axis_name="s")


_SC_PARAMS = pltpu.CompilerParams(use_tc_tiling_on_sc=False,
                                  needs_layout_passes=False)


def _splat(vec, lane):
  idx = jnp.full((16,), lane, jnp.int32)
  return jnp.take_along_axis(vec, idx, axis=0)


_PERM = []
for _j in range(4):
  for _k in range(16):
    _PERM.extend([32 * _j + _k, 32 * _j + 16 + _k])
_PERM = tuple(_PERM)


def _perm_cast(x):
  return jnp.take(x, jnp.array(_PERM, jnp.int32), axis=1).astype(jnp.bfloat16)


def _copies(tab, n):
  return tuple(
      jnp.concatenate([tab, jnp.zeros(((kk + 1) * 8,) + tab.shape[1:],
                                      tab.dtype)])
      for kk in range(n))


def _zero_vmem(ref, nrows, width):
  @pl.loop(0, nrows)
  def _(r):
    for j in range(width // 16):
      ref[r, pl.ds(16 * j, 16)] = jnp.zeros((16,), jnp.float32)


def _sc_spmm(x, gidx, sidx, adj):
  @functools.partial(
      pl.kernel,
      out_type=jax.ShapeDtypeStruct((2 * N_U, D), jnp.float32),
      mesh=_mesh(),
      compiler_params=_SC_PARAMS,
      scratch_types=[
          [pltpu.VMEM((CH,), jnp.int32) for _ in range(2)],
          [pltpu.VMEM((CH,), jnp.int32) for _ in range(2)],
          [pltpu.VMEM((CH,), jnp.float32) for _ in range(2)],
          [pltpu.VMEM((CH, D), jnp.bfloat16) for _ in range(2)],
          pltpu.VMEM((CH, D), jnp.float32),
          pltpu.VMEM_SHARED((N_U, D), jnp.float32),
          [pltpu.SemaphoreType.DMA for _ in range(2)],
          [pltpu.SemaphoreType.DMA for _ in range(2)],
      ],
  )
  def k(x0_hbm, x1_hbm, x2_hbm, x3_hbm, gidx_hbm, sidx_hbm, adj_hbm, out_hbm,
        gi, si, adj_v, rows, rows_f, acc, sem, isem):
    x_tabs = (x0_hbm, x1_hbm, x2_hbm, x3_hbm)
    c = lax.axis_index("c")
    s = lax.axis_index("s")
    _zero_vmem(rows_f, CH, D)
    stripe = s * 625
    for q in range(4):
      pltpu.sync_copy(rows_f, acc.at[pl.ds(stripe + q * CH, CH)])
    pltpu.sync_copy(rows_f.at[pl.ds(0, 113)],
                    acc.at[pl.ds(stripe + 4 * CH, 113)])
    plsc.subcore_barrier()

    cbase = c * SP_LEN + s * SP_PER_TILE
    pltpu.sync_copy(gidx_hbm.at[pl.ds(cbase, CH)], gi[0])
    pltpu.sync_copy(sidx_hbm.at[pl.ds(cbase, CH)], si[0])
    pltpu.sync_copy(adj_hbm.at[pl.ds(cbase, CH)], adj_v[0])
    for kk in range(4):
      pltpu.async_copy(x_tabs[kk].at[gi[0].at[pl.ds(32 * kk, 32)]],
                       rows[0].at[pl.ds(32 * kk, 32)], sem[0])
    pltpu.async_copy(gidx_hbm.at[pl.ds(cbase + CH, CH)], gi[1], isem[1])
    pltpu.async_copy(sidx_hbm.at[pl.ds(cbase + CH, CH)], si[1], isem[1])
    pltpu.async_copy(adj_hbm.at[pl.ds(cbase + CH, CH)], adj_v[1], isem[1])

    @pl.loop(0, SP_CHUNKS // 2)
    def _(g2):
      for b in range(2):
        nb = 1 - b
        ci = 2 * g2 + b

        @pl.when(ci + 1 < SP_CHUNKS)
        def _():
          pltpu.make_async_copy(gidx_hbm.at[pl.ds(0, CH)], gi[nb],
                                isem[nb]).wait()
          pltpu.make_async_copy(sidx_hbm.at[pl.ds(0, CH)], si[nb],
                                isem[nb]).wait()
          pltpu.make_async_copy(adj_hbm.at[pl.ds(0, CH)], adj_v[nb],
                                isem[nb]).wait()
          for kk in range(4):
            pltpu.async_copy(x_tabs[kk].at[gi[nb].at[pl.ds(32 * kk, 32)]],
                             rows[nb].at[pl.ds(32 * kk, 32)], sem[nb])

        for kk in range(4):
          pltpu.make_async_copy(x0_hbm.at[pl.ds(0, 32)],
                                rows[b].at[pl.ds(32 * kk, 32)], sem[b]).wait()

        @pl.loop(0, CH // 16)
        def _(q):
          av16 = adj_v[b][pl.ds(16 * q, 16)]
          for r in range(16):
            av = _splat(av16, r)
            rr = 16 * q + r
            for j in range(D // 32):
              v32 = rows[b][rr, pl.ds(32 * j, 32)]
              lo, hi = plsc.unpack(v32, format=plsc.PackFormat.INTERLEAVED)
              rows_f[rr, pl.ds(32 * j, 16)] = lo * av
              rows_f[rr, pl.ds(32 * j + 16, 16)] = hi * av

        pltpu.sync_copy(rows_f, acc.at[si[b]], add=True)

        @pl.when(ci + 2 < SP_CHUNKS)
        def _():
          b2 = cbase + (ci + 2) * CH
          pltpu.async_copy(gidx_hbm.at[pl.ds(b2, CH)], gi[b], isem[b])
          pltpu.async_copy(sidx_hbm.at[pl.ds(b2, CH)], si[b], isem[b])
          pltpu.async_copy(adj_hbm.at[pl.ds(b2, CH)], adj_v[b], isem[b])

    plsc.subcore_barrier()
    for q in range(4):
      pltpu.sync_copy(acc.at[pl.ds(stripe + q * CH, CH)],
                      out_hbm.at[pl.ds(c * N_U + stripe + q * CH, CH)])
    pltpu.sync_copy(acc.at[pl.ds(stripe + 4 * CH, 113)],
                    out_hbm.at[pl.ds(c * N_U + stripe + 4 * CH, 113)])

  return k(*x, gidx, sidx, adj)


def _sc_gat(h, elT, erT, src, dstg, dstl):
  @functools.partial(
      pl.kernel,
      out_type=(jax.ShapeDtypeStruct((2 * NLOC, D), jnp.float32),
                jax.ShapeDtypeStruct((2 * NLOC, 16), jnp.float32)),
      mesh=_mesh(),
      compiler_params=_SC_PARAMS,
      scratch_types=[
          [pltpu.VMEM((GCH,), jnp.int32) for _ in range(2)],
          [pltpu.VMEM((GCH,), jnp.int32) for _ in range(2)],
          [pltpu.VMEM((GCH,), jnp.int32) for _ in range(2)],
          [pltpu.VMEM((GCH, 16), jnp.float32) for _ in range(2)],
          [pltpu.VMEM((GCH, 16), jnp.float32) for _ in range(2)],
          [pltpu.VMEM((GCH, D), jnp.bfloat16) for _ in range(2)],
          pltpu.VMEM((GCH, D), jnp.float32),
          pltpu.VMEM((GCH, 16), jnp.float32),
          pltpu.VMEM_SHARED((NLOC, D), jnp.float32),
          pltpu.VMEM_SHARED((NLOC, 16), jnp.float32),
          [pltpu.SemaphoreType.DMA for _ in range(2)],
          [pltpu.SemaphoreType.DMA for _ in range(2)],
      ],
  )
  def k(h_hbm, h2_hbm, elTa_hbm, elTb_hbm, erTa_hbm, erTb_hbm,
        src_hbm, dstg_hbm, dstl_hbm, raw_hbm, s_hbm,
        si, dg, di, el_s, er_d, hrows, hrows_f, trows, acc, sacc, sem, isem):
    h_tabs = (h_hbm, h2_hbm)
    el_tabs = (elTa_hbm, elTb_hbm)
    er_tabs = (erTa_hbm, erTb_hbm)
    c = lax.axis_index("c")
    s = lax.axis_index("s")
    _zero_vmem(hrows_f, GCH, D)
    _zero_vmem(trows, GCH, 16)
    stripe = s * STRIPE
    nfull, rem = STRIPE // GCH, STRIPE % GCH
    for q in range(nfull):
      pltpu.sync_copy(hrows_f, acc.at[pl.ds(stripe + q * GCH, GCH)])
      pltpu.sync_copy(trows, sacc.at[pl.ds(stripe + q * GCH, GCH)])
    if rem:
      pltpu.sync_copy(hrows_f.at[pl.ds(0, rem)],
                      acc.at[pl.ds(stripe + nfull * GCH, rem)])
      pltpu.sync_copy(trows.at[pl.ds(0, rem)],
                      sacc.at[pl.ds(stripe + nfull * GCH, rem)])
    plsc.subcore_barrier()

    cbase = c * GA_LEN + s * GA_PER_TILE
    pltpu.sync_copy(src_hbm.at[pl.ds(cbase, GCH)], si[0])
    pltpu.sync_copy(dstg_hbm.at[pl.ds(cbase, GCH)], dg[0])
    pltpu.sync_copy(dstl_hbm.at[pl.ds(cbase, GCH)], di[0])
    for kk in range(2):
      pltpu.async_copy(el_tabs[kk].at[si[0].at[pl.ds(48 * kk, 48)]],
                       el_s[0].at[pl.ds(48 * kk, 48)], sem[0])
      pltpu.async_copy(er_tabs[kk].at[dg[0].at[pl.ds(48 * kk, 48)]],
                       er_d[0].at[pl.ds(48 * kk, 48)], sem[0])
      pltpu.async_copy(h_tabs[kk].at[si[0].at[pl.ds(48 * kk, 48)]],
                       hrows[0].at[pl.ds(48 * kk, 48)], sem[0])
    pltpu.async_copy(src_hbm.at[pl.ds(cbase + GCH, GCH)], si[1], isem[1])
    pltpu.async_copy(dstg_hbm.at[pl.ds(cbase + GCH, GCH)], dg[1], isem[1])
    pltpu.async_copy(dstl_hbm.at[pl.ds(cbase + GCH, GCH)], di[1], isem[1])

    @pl.loop(0, GA_CHUNKS // 2)
    def _(g2):
      for b in range(2):
        nb = 1 - b
        ci = 2 * g2 + b

        @pl.when(ci + 1 < GA_CHUNKS)
        def _():
          pltpu.make_async_copy(src_hbm.at[pl.ds(0, GCH)], si[nb],
                                isem[nb]).wait()
          pltpu.make_async_copy(src_hbm.at[pl.ds(0, GCH)], dg[nb],
                                isem[nb]).wait()
          pltpu.make_async_copy(src_hbm.at[pl.ds(0, GCH)], di[nb],
                                isem[nb]).wait()
          for kk in range(2):
            pltpu.async_copy(el_tabs[kk].at[si[nb].at[pl.ds(48 * kk, 48)]],
                             el_s[nb].at[pl.ds(48 * kk, 48)], sem[nb])
            pltpu.async_copy(er_tabs[kk].at[dg[nb].at[pl.ds(48 * kk, 48)]],
                             er_d[nb].at[pl.ds(48 * kk, 48)], sem[nb])
            pltpu.async_copy(h_tabs[kk].at[si[nb].at[pl.ds(48 * kk, 48)]],
                             hrows[nb].at[pl.ds(48 * kk, 48)], sem[nb])

        for kk in range(2):
          pltpu.make_async_copy(elTa_hbm.at[pl.ds(0, 48)],
                                el_s[b].at[pl.ds(48 * kk, 48)], sem[b]).wait()
          pltpu.make_async_copy(elTa_hbm.at[pl.ds(0, 48)],
                                er_d[b].at[pl.ds(48 * kk, 48)], sem[b]).wait()
          pltpu.make_async_copy(h_hbm.at[pl.ds(0, 48)],
                                hrows[b].at[pl.ds(48 * kk, 48)], sem[b]).wait()

        @pl.loop(0, GCH)
        def _(r):
          v = el_s[b][r, :] + er_d[b][r, :]
          t = jnp.exp(jnp.maximum(v, v * NEG_SLOPE))
          trows[r, :] = t
          for hh in range(H):
            tv = _splat(t, hh)
            v32 = hrows[b][r, pl.ds(DH * hh, 32)]
            lo, hi = plsc.unpack(v32, format=plsc.PackFormat.INTERLEAVED)
            hrows_f[r, pl.ds(DH * hh, 16)] = lo * tv
            hrows_f[r, pl.ds(DH * hh + 16, 16)] = hi * tv

        pltpu.sync_copy(trows, sacc.at[di[b]], add=True)
        pltpu.sync_copy(hrows_f, acc.at[di[b]], add=True)

        @pl.when(ci + 2 < GA_CHUNKS)
        def _():
          b2 = cbase + (ci + 2) * GCH
          pltpu.async_copy(src_hbm.at[pl.ds(b2, GCH)], si[b], isem[b])
          pltpu.async_copy(dstg_hbm.at[pl.ds(b2, GCH)], dg[b], isem[b])
          pltpu.async_copy(dstl_hbm.at[pl.ds(b2, GCH)], di[b], isem[b])

    plsc.subcore_barrier()
    for q in range(nfull):
      pltpu.sync_copy(acc.at[pl.ds(stripe + q * GCH, GCH)],
                      raw_hbm.at[pl.ds(c * NLOC + stripe + q * GCH, GCH)])
      pltpu.sync_copy(sacc.at[pl.ds(stripe + q * GCH, GCH)],
                      s_hbm.at[pl.ds(c * NLOC + stripe + q * GCH, GCH)])
    if rem:
      off = stripe + nfull * GCH
      pltpu.sync_copy(acc.at[pl.ds(off, rem)],
                      raw_hbm.at[pl.ds(c * NLOC + off, rem)])
      pltpu.sync_copy(sacc.at[pl.ds(off, rem)],
                      s_hbm.at[pl.ds(c * NLOC + off, rem)])

  return k(*h, *elT, *erT, src, dstg, dstl)


def _sc_batch(gsum, esum, idxg, idxe):
  @functools.partial(
      pl.kernel,
      out_type=jax.ShapeDtypeStruct((6 * B, D), jnp.float32),
      mesh=_mesh(),
      compiler_params=_SC_PARAMS,
      scratch_types=[
          pltpu.VMEM((CH,), jnp.int32),
          pltpu.VMEM((CH, D), jnp.float32),
          pltpu.SemaphoreType.DMA,
      ],
  )
  def k(g_hbm, e_hbm, idxg_hbm, idxe_hbm, out_hbm, idx_v, rows, sem):
    c = lax.axis_index("c")
    s = lax.axis_index("s")

    @pl.when(c == 0)
    def _():
      pltpu.sync_copy(idxg_hbm.at[pl.ds(s * CH, CH)], idx_v)
      pltpu.async_copy(g_hbm.at[idx_v], rows, sem).wait()
      pltpu.sync_copy(rows, out_hbm.at[pl.ds(s * CH, CH)])

    @pl.when(c == 1)
    def _():
      for q in range(2):
        pltpu.sync_copy(idxe_hbm.at[pl.ds(s * 2 * CH + q * CH, CH)], idx_v)
        pltpu.async_copy(e_hbm.at[idx_v], rows, sem).wait()
        pltpu.sync_copy(rows, out_hbm.at[pl.ds(2 * B + s * 2 * CH + q * CH, CH)])

  return k(gsum, esum, idxg, idxe)


_RB = 2000


def _tc_pre(x, W, wL, wR):
  def body(x_ref, w_ref, wl_ref, wr_ref, h_ref, el_ref, er_ref):
    xb = x_ref[...]
    h_ref[...] = jnp.dot(xb, w_ref[...], preferred_element_type=jnp.float32)
    el_ref[...] = jnp.dot(xb, wl_ref[...], preferred_element_type=jnp.float32)
    er_ref[...] = jnp.dot(xb, wr_ref[...], preferred_element_type=jnp.float32)

  grid = N // _RB
  return pl.pallas_call(
      body,
      grid=(grid,),
      in_specs=[
          pl.BlockSpec((_RB, D), lambda i: (i, 0)),
          pl.BlockSpec((D, D), lambda i: (0, 0)),
          pl.BlockSpec((D, 16), lambda i: (0, 0)),
          pl.BlockSpec((D, 16), lambda i: (0, 0)),
      ],
      out_specs=[
          pl.BlockSpec((_RB, D), lambda i: (i, 0)),
          pl.BlockSpec((_RB, 16), lambda i: (i, 0)),
          pl.BlockSpec((_RB, 16), lambda i: (i, 0)),
      ],
      out_shape=[
          jax.ShapeDtypeStruct((N, D), jnp.float32),
          jax.ShapeDtypeStruct((N, 16), jnp.float32),
          jax.ShapeDtypeStruct((N, 16), jnp.float32),
      ],
  )(x, W, wL, wR)


def _gat_norm(raw, sden):
  n = raw.shape[0]
  den = sden[:, :H] + 1e-9
  g3 = raw.reshape(n, H, DH) / den[:, :, None]
  g = g3.reshape(n, D)
  return jnp.where(g > 0, g, jnp.exp(g) - 1.0)


def _tc_mid(x0, Z1, raw1, s1, W, wL, wR):
  def body(x0_ref, z_ref, raw_ref, s_ref, w_ref, wl_ref, wr_ref,
           x1_ref, gacc_ref, h_ref, el_ref, er_ref):
    x0b = x0_ref[...]
    x1b = x0b + z_ref[...]
    x1_ref[...] = x1b
    gacc_ref[...] = x0b + _gat_norm(raw_ref[...], s_ref[...])
    h_ref[...] = jnp.dot(x1b, w_ref[...], preferred_element_type=jnp.float32)
    el_ref[...] = jnp.dot(x1b, wl_ref[...], preferred_element_type=jnp.float32)
    er_ref[...] = jnp.dot(x1b, wr_ref[...], preferred_element_type=jnp.float32)

  grid = N // _RB
  return pl.pallas_call(
      body,
      grid=(grid,),
      in_specs=[
          pl.BlockSpec((_RB, D), lambda i: (i, 0)),
          pl.BlockSpec((_RB, D), lambda i: (i, 0)),
          pl.BlockSpec((_RB, D), lambda i: (i, 0)),
          pl.BlockSpec((_RB, 16), lambda i: (i, 0)),
          pl.BlockSpec((D, D), lambda i: (0, 0)),
          pl.BlockSpec((D, 16), lambda i: (0, 0)),
          pl.BlockSpec((D, 16), lambda i: (0, 0)),
      ],
      out_specs=[
          pl.BlockSpec((_RB, D), lambda i: (i, 0)),
          pl.BlockSpec((_RB, D), lambda i: (i, 0)),
          pl.BlockSpec((_RB, D), lambda i: (i, 0)),
          pl.BlockSpec((_RB, 16), lambda i: (i, 0)),
          pl.BlockSpec((_RB, 16), lambda i: (i, 0)),
      ],
      out_shape=[
          jax.ShapeDtypeStruct((N, D), jnp.float32),
          jax.ShapeDtypeStruct((N, D), jnp.float32),
          jax.ShapeDtypeStruct((N, D), jnp.float32),
          jax.ShapeDtypeStruct((N, 16), jnp.float32),
          jax.ShapeDtypeStruct((N, 16), jnp.float32),
      ],
  )(x0, Z1, raw1, s1, W, wL, wR)


def _tc_final(x0, x1, Z2, Gacc, raw2, s2):
  def body(x0_ref, x1_ref, z_ref, gacc_ref, raw_ref, s_ref,
           e_ref, g_ref, reg_ref):
    i = pl.program_id(0)
    x0b = x0_ref[...]
    e_ref[...] = x0b + 2.0 * x1_ref[...] + z_ref[...]
    g_ref[...] = gacc_ref[...] + _gat_norm(raw_ref[...], s_ref[...])

    @pl.when(i == 0)
    def _():
      reg_ref[...] = jnp.zeros((1, 1), jnp.float32)

    reg_ref[...] += jnp.reshape(jnp.sum(x0b * x0b), (1, 1))

  grid = N // _RB
  return pl.pallas_call(
      body,
      grid=(grid,),
      in_specs=[
          pl.BlockSpec((_RB, D), lambda i: (i, 0)),
          pl.BlockSpec((_RB, D), lambda i: (i, 0)),
          pl.BlockSpec((_RB, D), lambda i: (i, 0)),
          pl.BlockSpec((_RB, D), lambda i: (i, 0)),
          pl.BlockSpec((_RB, D), lambda i: (i, 0)),
          pl.BlockSpec((_RB, 16), lambda i: (i, 0)),
      ],
      out_specs=[
          pl.BlockSpec((_RB, D), lambda i: (i, 0)),
          pl.BlockSpec((_RB, D), lambda i: (i, 0)),
          pl.BlockSpec((1, 1), lambda i: (0, 0)),
      ],
      out_shape=[
          jax.ShapeDtypeStruct((N, D), jnp.float32),
          jax.ShapeDtypeStruct((N, D), jnp.float32),
          jax.ShapeDtypeStruct((1, 1), jnp.float32),
      ],
  )(x0, x1, Z2, Gacc, raw2, s2)


_CB = 1000


def _tc_loss(batch, esum, regsum):
  nsteps = N_U // _CB

  def body(batch_ref, eu_ref, ei_ref, reg_ref,
           su_ref, si_ref, loss_ref, lr_ref, ls_ref):
    k = pl.program_id(0)

    @pl.when(k == 0)
    def _():
      su_ref[...] = jnp.zeros((B,), jnp.float32)
      si_ref[...] = jnp.zeros((B,), jnp.float32)

    gu = batch_ref[0:B, :]
    gi = batch_ref[B:2 * B, :]
    dn = (((1,), (1,)), ((), ()))
    lu = lax.dot_general(gu, eu_ref[...], dn,
                         preferred_element_type=jnp.float32)
    li = lax.dot_general(gi, ei_ref[...], dn,
                         preferred_element_type=jnp.float32)
    su_ref[...] += jnp.sum(jnp.exp(lu * (1.0 / TEMP)), axis=1)
    si_ref[...] += jnp.sum(jnp.exp(li * (1.0 / TEMP)), axis=1)

    @pl.when(k == nsteps - 1)
    def _():
      eu = batch_ref[2 * B:3 * B, :]
      ei = batch_ref[3 * B:4 * B, :]
      posb = batch_ref[4 * B:5 * B, :]
      negb = batch_ref[5 * B:6 * B, :]
      neg_score = (jnp.mean(jnp.log(su_ref[...] + 1e-8)) +
                   jnp.mean(jnp.log(si_ref[...] + 1e-8)))
      pos_score = (
          jnp.mean(jnp.log(jnp.exp(jnp.sum(gu * eu, axis=1) * (1.0 / TEMP)))) +
          jnp.mean(jnp.log(jnp.exp(jnp.sum(gi * ei, axis=1) * (1.0 / TEMP)))))
      loss_s = -pos_score + neg_score
      pos_sc = jnp.sum(eu * posb, axis=1)
      neg_sc = jnp.sum(eu * negb, axis=1)
      loss_r = -jnp.mean(jnp.log(jax.nn.sigmoid(pos_sc - neg_sc)))
      lr_ref[...] = jnp.reshape(loss_r, (1, 1))
      ls_ref[...] = jnp.reshape(LAMBDA_1 * loss_s, (1, 1))
      loss_ref[...] = (jnp.reshape(loss_r + LAMBDA_1 * loss_s, (1, 1)) +
                       reg_ref[...] * LAMBDA_2)

  return pl.pallas_call(
      body,
      grid=(nsteps,),
      in_specs=[
          pl.BlockSpec((6 * B, D), lambda k: (0, 0)),
          pl.BlockSpec((_CB, D), lambda k: (k, 0)),
          pl.BlockSpec((_CB, D), lambda k: (k + N_U // _CB, 0)),
          pl.BlockSpec((1, 1), lambda k: (0, 0)),
      ],
      out_specs=[
          pl.BlockSpec((B,), lambda k: (0,)),
          pl.BlockSpec((B,), lambda k: (0,)),
          pl.BlockSpec((1, 1), lambda k: (0, 0)),
          pl.BlockSpec((1, 1), lambda k: (0, 0)),
          pl.BlockSpec((1, 1), lambda k: (0, 0)),
      ],
      out_shape=[
          jax.ShapeDtypeStruct((B,), jnp.float32),
          jax.ShapeDtypeStruct((B,), jnp.float32),
          jax.ShapeDtypeStruct((1, 1), jnp.float32),
          jax.ShapeDtypeStruct((1, 1), jnp.float32),
          jax.ShapeDtypeStruct((1, 1), jnp.float32),
      ],
  )(batch, esum, esum, regsum)


def kernel(uids, iids, pos, neg, E_u_0, E_i_0, W_gat, attn_l, attn_r,
           adj_vals, edge_u, edge_i):
  i32 = jnp.int32
  uids = uids.astype(i32)
  iids = iids.astype(i32)
  pos = pos.astype(i32)
  neg = neg.astype(i32)
  edge_u = edge_u.astype(i32)
  edge_i = edge_i.astype(i32)
  adj_vals = adj_vals.astype(jnp.float32)

  x0 = jnp.concatenate([E_u_0, E_i_0], axis=0)
  W = W_gat.reshape(D, H * DH)
  wl = jnp.einsum("dhk,hk->dh", W_gat, attn_l)
  wr = jnp.einsum("dhk,hk->dh", W_gat, attn_r)
  wL = jnp.pad(wl, ((0, 0), (0, 16 - H)))
  wR = jnp.pad(wr, ((0, 0), (0, 16 - H)))

  padS = jnp.zeros((SP_LEN - E,), i32)
  padSf = jnp.zeros((SP_LEN - E,), jnp.float32)
  gidx = jnp.concatenate([edge_i + N_U, padS, edge_u, padS])
  sidx = jnp.concatenate([edge_u, padS, edge_i, padS])
  adj2 = jnp.concatenate([adj_vals, padSf, adj_vals, padSf])

  ar_u = jnp.arange(N_U, dtype=i32)
  npad = GA_LEN - (E + N_U)
  pad0 = jnp.zeros((npad,), i32)
  padD = jnp.full((npad,), N_U, i32)
  src = jnp.concatenate([edge_i + N_U, ar_u, pad0, edge_u, ar_u + N_U, pad0])
  dstg = jnp.concatenate([edge_u, ar_u, pad0, edge_i + N_U, ar_u + N_U, pad0])
  dstl = jnp.concatenate([edge_u, ar_u, padD, edge_i, ar_u, padD])

  idxg = jnp.concatenate([uids, iids + N_U])
  idxe = jnp.concatenate([uids, iids + N_U, pos + N_U, neg + N_U])

  h1, elT1, erT1 = _tc_pre(x0, W, wL, wR)
  Z1 = _sc_spmm(_copies(_perm_cast(x0), 4), gidx, sidx, adj2)
  raw1, s1 = _sc_gat(_copies(_perm_cast(h1), 2), _copies(elT1, 2),
                     _copies(erT1, 2), src, dstg, dstl)
  raw1c = jnp.concatenate([raw1[:N_U], raw1[NLOC:NLOC + N_I]])
  s1c = jnp.concatenate([s1[:N_U], s1[NLOC:NLOC + N_I]])
  x1, Gacc, h2, elT2, erT2 = _tc_mid(x0, Z1, raw1c, s1c, W, wL, wR)

  Z2 = _sc_spmm(_copies(_perm_cast(x1), 4), gidx, sidx, adj2)
  raw2, s2 = _sc_gat(_copies(_perm_cast(h2), 2), _copies(elT2, 2),
                     _copies(erT2, 2), src, dstg, dstl)
  raw2c = jnp.concatenate([raw2[:N_U], raw2[NLOC:NLOC + N_I]])
  s2c = jnp.concatenate([s2[:N_U], s2[NLOC:NLOC + N_I]])
  esum, gsum, regsum = _tc_final(x0, x1, Z2, Gacc, raw2c, s2c)

  batch = _sc_batch(gsum, esum, idxg, idxe)
  _, _, loss, lr, ls = _tc_loss(batch, esum, regsum)
  return (loss[0, 0], lr[0, 0], ls[0, 0])


# speedup vs baseline: 1.0896x; 1.0896x over previous
import functools
import jax
import jax.numpy as jnp
from jax import lax
from jax.experimental import pallas as pl
from jax.experimental.pallas import tpu as pltpu
from jax.experimental.pallas import tpu_sc as plsc

N_U = 10000
N_I = 10000
N = N_U + N_I
D = 128
E = 160000
H = 4
DH = 32
TEMP = 0.2
LAMBDA_1 = 0.2
LAMBDA_2 = 1e-07
NEG_SLOPE = 0.2
B = 1024

NS = 16
CH = 128

SP_PER_TILE = 10240
SP_CHUNKS = SP_PER_TILE // CH
SP_LEN = NS * SP_PER_TILE

GCH = 96
GA_PER_TILE = 10752
GA_CHUNKS = GA_PER_TILE // GCH
GA_LEN = NS * GA_PER_TILE
NLOC = 10016
STRIPE = NLOC // NS


def _mesh():
  return plsc.VectorSubcoreMesh(core_axis_name="c", subcore_axis_name="s")


_SC_PARAMS = pltpu.CompilerParams(use_tc_tiling_on_sc=False)


def _splat(vec, lane):
  idx = jnp.full((16,), lane, jnp.int32)
  return jnp.take_along_axis(vec, idx, axis=0)


def _zero_vmem(ref, nrows, width):
  @pl.loop(0, nrows)
  def _(r):
    for j in range(width // 16):
      ref[r, pl.ds(16 * j, 16)] = jnp.zeros((16,), jnp.float32)


def _sc_spmm(x, gidx, sidx, adj):
  @functools.partial(
      pl.kernel,
      out_type=jax.ShapeDtypeStruct((2 * N_U, D), jnp.float32),
      mesh=_mesh(),
      compiler_params=_SC_PARAMS,
      scratch_types=[
          [pltpu.VMEM((CH,), jnp.int32) for _ in range(2)],
          [pltpu.VMEM((CH,), jnp.int32) for _ in range(2)],
          [pltpu.VMEM((CH,), jnp.float32) for _ in range(2)],
          [pltpu.VMEM((CH, D), jnp.float32) for _ in range(2)],
          pltpu.VMEM_SHARED((N_U, D), jnp.float32),
          [pltpu.SemaphoreType.DMA for _ in range(2)],
          [pltpu.SemaphoreType.DMA for _ in range(2)],
      ],
  )
  def k(x_hbm, gidx_hbm, sidx_hbm, adj_hbm, out_hbm,
        gi, si, adj_v, rows, acc, sem, isem):
    c = lax.axis_index("c")
    s = lax.axis_index("s")
    _zero_vmem(rows[0], CH, D)
    stripe = s * 625
    for q in range(4):
      pltpu.sync_copy(rows[0], acc.at[pl.ds(stripe + q * CH, CH)])
    pltpu.sync_copy(rows[0].at[pl.ds(0, 113)],
                    acc.at[pl.ds(stripe + 4 * CH, 113)])
    plsc.subcore_barrier()

    cbase = c * SP_LEN + s * SP_PER_TILE
    pltpu.sync_copy(gidx_hbm.at[pl.ds(cbase, CH)], gi[0])
    pltpu.sync_copy(sidx_hbm.at[pl.ds(cbase, CH)], si[0])
    pltpu.sync_copy(adj_hbm.at[pl.ds(cbase, CH)], adj_v[0])
    pltpu.async_copy(x_hbm.at[gi[0]], rows[0], sem[0])
    pltpu.async_copy(gidx_hbm.at[pl.ds(cbase + CH, CH)], gi[1], isem[1])
    pltpu.async_copy(sidx_hbm.at[pl.ds(cbase + CH, CH)], si[1], isem[1])
    pltpu.async_copy(adj_hbm.at[pl.ds(cbase + CH, CH)], adj_v[1], isem[1])

    @pl.loop(0, SP_CHUNKS // 2)
    def _(g2):
      for b in range(2):
        nb = 1 - b
        ci = 2 * g2 + b

        @pl.when(ci + 1 < SP_CHUNKS)
        def _():
          pltpu.make_async_copy(gidx_hbm.at[pl.ds(0, CH)], gi[nb],
                                isem[nb]).wait()
          pltpu.make_async_copy(sidx_hbm.at[pl.ds(0, CH)], si[nb],
                                isem[nb]).wait()
          pltpu.make_async_copy(adj_hbm.at[pl.ds(0, CH)], adj_v[nb],
                                isem[nb]).wait()
          pltpu.async_copy(x_hbm.at[gi[nb]], rows[nb], sem[nb])

        pltpu.make_async_copy(x_hbm.at[pl.ds(0, CH)], rows[b], sem[b]).wait()

        @pl.loop(0, CH // 16)
        def _(q):
          av16 = adj_v[b][pl.ds(16 * q, 16)]
          for r in range(16):
            av = _splat(av16, r)
            for j in range(D // 16):
              rows[b][16 * q + r, pl.ds(16 * j, 16)] = (
                  rows[b][16 * q + r, pl.ds(16 * j, 16)] * av)

        pltpu.sync_copy(rows[b], acc.at[si[b]], add=True)

        @pl.when(ci + 2 < SP_CHUNKS)
        def _():
          b2 = cbase + (ci + 2) * CH
          pltpu.async_copy(gidx_hbm.at[pl.ds(b2, CH)], gi[b], isem[b])
          pltpu.async_copy(sidx_hbm.at[pl.ds(b2, CH)], si[b], isem[b])
          pltpu.async_copy(adj_hbm.at[pl.ds(b2, CH)], adj_v[b], isem[b])

    plsc.subcore_barrier()
    for q in range(4):
      pltpu.sync_copy(acc.at[pl.ds(stripe + q * CH, CH)],
                      out_hbm.at[pl.ds(c * N_U + stripe + q * CH, CH)])
    pltpu.sync_copy(acc.at[pl.ds(stripe + 4 * CH, 113)],
                    out_hbm.at[pl.ds(c * N_U + stripe + 4 * CH, 113)])

  return k(x, gidx, sidx, adj)


def _sc_gat(h, elT, erT, src, dstg, dstl):
  @functools.partial(
      pl.kernel,
      out_type=(jax.ShapeDtypeStruct((2 * NLOC, D), jnp.float32),
                jax.ShapeDtypeStruct((2 * NLOC, 16), jnp.float32)),
      mesh=_mesh(),
      compiler_params=_SC_PARAMS,
      scratch_types=[
          [pltpu.VMEM((GCH,), jnp.int32) for _ in range(2)],
          [pltpu.VMEM((GCH,), jnp.int32) for _ in range(2)],
          [pltpu.VMEM((GCH,), jnp.int32) for _ in range(2)],
          [pltpu.VMEM((GCH, 16), jnp.float32) for _ in range(2)],
          [pltpu.VMEM((GCH, 16), jnp.float32) for _ in range(2)],
          [pltpu.VMEM((GCH, D), jnp.float32) for _ in range(2)],
          [pltpu.VMEM((GCH, 16), jnp.float32) for _ in range(2)],
          pltpu.VMEM_SHARED((NLOC, D), jnp.float32),
          pltpu.VMEM_SHARED((NLOC, 16), jnp.float32),
          [pltpu.SemaphoreType.DMA for _ in range(2)],
          [pltpu.SemaphoreType.DMA for _ in range(2)],
      ],
  )
  def k(h_hbm, elT_hbm, erT_hbm, src_hbm, dstg_hbm, dstl_hbm, raw_hbm, s_hbm,
        si, dg, di, el_s, er_d, hrows, trows, acc, sacc, sem, isem):
    c = lax.axis_index("c")
    s = lax.axis_index("s")
    _zero_vmem(hrows[0], GCH, D)
    _zero_vmem(trows[0], GCH, 16)
    stripe = s * STRIPE
    nfull, rem = STRIPE // GCH, STRIPE % GCH
    for q in range(nfull):
      pltpu.sync_copy(hrows[0], acc.at[pl.ds(stripe + q * GCH, GCH)])
      pltpu.sync_copy(trows[0], sacc.at[pl.ds(stripe + q * GCH, GCH)])
    if rem:
      pltpu.sync_copy(hrows[0].at[pl.ds(0, rem)],
                      acc.at[pl.ds(stripe + nfull * GCH, rem)])
      pltpu.sync_copy(trows[0].at[pl.ds(0, rem)],
                      sacc.at[pl.ds(stripe + nfull * GCH, rem)])
    plsc.subcore_barrier()

    cbase = c * GA_LEN + s * GA_PER_TILE
    pltpu.sync_copy(src_hbm.at[pl.ds(cbase, GCH)], si[0])
    pltpu.sync_copy(dstg_hbm.at[pl.ds(cbase, GCH)], dg[0])
    pltpu.sync_copy(dstl_hbm.at[pl.ds(cbase, GCH)], di[0])
    pltpu.async_copy(elT_hbm.at[si[0]], el_s[0], sem[0])
    pltpu.async_copy(erT_hbm.at[dg[0]], er_d[0], sem[0])
    pltpu.async_copy(h_hbm.at[si[0]], hrows[0], sem[0])
    pltpu.async_copy(src_hbm.at[pl.ds(cbase + GCH, GCH)], si[1], isem[1])
    pltpu.async_copy(dstg_hbm.at[pl.ds(cbase + GCH, GCH)], dg[1], isem[1])
    pltpu.async_copy(dstl_hbm.at[pl.ds(cbase + GCH, GCH)], di[1], isem[1])

    @pl.loop(0, GA_CHUNKS // 2)
    def _(g2):
      for b in range(2):
        nb = 1 - b
        ci = 2 * g2 + b

        @pl.when(ci + 1 < GA_CHUNKS)
        def _():
          pltpu.make_async_copy(src_hbm.at[pl.ds(0, GCH)], si[nb],
                                isem[nb]).wait()
          pltpu.make_async_copy(src_hbm.at[pl.ds(0, GCH)], dg[nb],
                                isem[nb]).wait()
          pltpu.make_async_copy(src_hbm.at[pl.ds(0, GCH)], di[nb],
                                isem[nb]).wait()
          pltpu.async_copy(elT_hbm.at[si[nb]], el_s[nb], sem[nb])
          pltpu.async_copy(erT_hbm.at[dg[nb]], er_d[nb], sem[nb])
          pltpu.async_copy(h_hbm.at[si[nb]], hrows[nb], sem[nb])

        pltpu.make_async_copy(elT_hbm.at[pl.ds(0, GCH)], el_s[b],
                              sem[b]).wait()
        pltpu.make_async_copy(elT_hbm.at[pl.ds(0, GCH)], er_d[b],
                              sem[b]).wait()
        pltpu.make_async_copy(h_hbm.at[pl.ds(0, GCH)], hrows[b],
                              sem[b]).wait()

        @pl.loop(0, GCH)
        def _(r):
          v = el_s[b][r, :] + er_d[b][r, :]
          t = jnp.exp(jnp.maximum(v, v * NEG_SLOPE))
          trows[b][r, :] = t
          for hh in range(H):
            tv = _splat(t, hh)
            hrows[b][r, pl.ds(DH * hh, 16)] = (
                hrows[b][r, pl.ds(DH * hh, 16)] * tv)
            hrows[b][r, pl.ds(DH * hh + 16, 16)] = (
                hrows[b][r, pl.ds(DH * hh + 16, 16)] * tv)

        pltpu.sync_copy(trows[b], sacc.at[di[b]], add=True)
        pltpu.sync_copy(hrows[b], acc.at[di[b]], add=True)

        @pl.when(ci + 2 < GA_CHUNKS)
        def _():
          b2 = cbase + (ci + 2) * GCH
          pltpu.async_copy(src_hbm.at[pl.ds(b2, GCH)], si[b], isem[b])
          pltpu.async_copy(dstg_hbm.at[pl.ds(b2, GCH)], dg[b], isem[b])
          pltpu.async_copy(dstl_hbm.at[pl.ds(b2, GCH)], di[b], isem[b])

    plsc.subcore_barrier()
    for q in range(nfull):
      pltpu.sync_copy(acc.at[pl.ds(stripe + q * GCH, GCH)],
                      raw_hbm.at[pl.ds(c * NLOC + stripe + q * GCH, GCH)])
      pltpu.sync_copy(sacc.at[pl.ds(stripe + q * GCH, GCH)],
                      s_hbm.at[pl.ds(c * NLOC + stripe + q * GCH, GCH)])
    if rem:
      off = stripe + nfull * GCH
      pltpu.sync_copy(acc.at[pl.ds(off, rem)],
                      raw_hbm.at[pl.ds(c * NLOC + off, rem)])
      pltpu.sync_copy(sacc.at[pl.ds(off, rem)],
                      s_hbm.at[pl.ds(c * NLOC + off, rem)])

  return k(h, elT, erT, src, dstg, dstl)


def _sc_batch(gsum, esum, idxg, idxe):
  @functools.partial(
      pl.kernel,
      out_type=jax.ShapeDtypeStruct((6 * B, D), jnp.float32),
      mesh=_mesh(),
      compiler_params=_SC_PARAMS,
      scratch_types=[
          pltpu.VMEM((CH,), jnp.int32),
          pltpu.VMEM((CH, D), jnp.float32),
          pltpu.SemaphoreType.DMA,
      ],
  )
  def k(g_hbm, e_hbm, idxg_hbm, idxe_hbm, out_hbm, idx_v, rows, sem):
    c = lax.axis_index("c")
    s = lax.axis_index("s")

    @pl.when(c == 0)
    def _():
      pltpu.sync_copy(idxg_hbm.at[pl.ds(s * CH, CH)], idx_v)
      pltpu.async_copy(g_hbm.at[idx_v], rows, sem).wait()
      pltpu.sync_copy(rows, out_hbm.at[pl.ds(s * CH, CH)])

    @pl.when(c == 1)
    def _():
      for q in range(2):
        pltpu.sync_copy(idxe_hbm.at[pl.ds(s * 2 * CH + q * CH, CH)], idx_v)
        pltpu.async_copy(e_hbm.at[idx_v], rows, sem).wait()
        pltpu.sync_copy(rows, out_hbm.at[pl.ds(2 * B + s * 2 * CH + q * CH, CH)])

  return k(gsum, esum, idxg, idxe)


_RB = 2000


def _tc_pre(x, W, wL, wR):
  def body(x_ref, w_ref, wl_ref, wr_ref, h_ref, el_ref, er_ref):
    xb = x_ref[...]
    h_ref[...] = jnp.dot(xb, w_ref[...], preferred_element_type=jnp.float32)
    el_ref[...] = jnp.dot(xb, wl_ref[...], preferred_element_type=jnp.float32)
    er_ref[...] = jnp.dot(xb, wr_ref[...], preferred_element_type=jnp.float32)

  grid = N // _RB
  return pl.pallas_call(
      body,
      grid=(grid,),
      in_specs=[
          pl.BlockSpec((_RB, D), lambda i: (i, 0)),
          pl.BlockSpec((D, D), lambda i: (0, 0)),
          pl.BlockSpec((D, 16), lambda i: (0, 0)),
          pl.BlockSpec((D, 16), lambda i: (0, 0)),
      ],
      out_specs=[
          pl.BlockSpec((_RB, D), lambda i: (i, 0)),
          pl.BlockSpec((_RB, 16), lambda i: (i, 0)),
          pl.BlockSpec((_RB, 16), lambda i: (i, 0)),
      ],
      out_shape=[
          jax.ShapeDtypeStruct((N, D), jnp.float32),
          jax.ShapeDtypeStruct((N, 16), jnp.float32),
          jax.ShapeDtypeStruct((N, 16), jnp.float32),
      ],
  )(x, W, wL, wR)


def _gat_norm(raw, sden):
  n = raw.shape[0]
  den = sden[:, :H] + 1e-9
  g3 = raw.reshape(n, H, DH) / den[:, :, None]
  g = g3.reshape(n, D)
  return jnp.where(g > 0, g, jnp.exp(g) - 1.0)


def _tc_mid(x0, Z1, raw1, s1, W, wL, wR):
  def body(x0_ref, z_ref, raw_ref, s_ref, w_ref, wl_ref, wr_ref,
           x1_ref, gacc_ref, h_ref, el_ref, er_ref):
    x0b = x0_ref[...]
    x1b = x0b + z_ref[...]
    x1_ref[...] = x1b
    gacc_ref[...] = x0b + _gat_norm(raw_ref[...], s_ref[...])
    h_ref[...] = jnp.dot(x1b, w_ref[...], preferred_element_type=jnp.float32)
    el_ref[...] = jnp.dot(x1b, wl_ref[...], preferred_element_type=jnp.float32)
    er_ref[...] = jnp.dot(x1b, wr_ref[...], preferred_element_type=jnp.float32)

  grid = N // _RB
  return pl.pallas_call(
      body,
      grid=(grid,),
      in_specs=[
          pl.BlockSpec((_RB, D), lambda i: (i, 0)),
          pl.BlockSpec((_RB, D), lambda i: (i, 0)),
          pl.BlockSpec((_RB, D), lambda i: (i, 0)),
          pl.BlockSpec((_RB, 16), lambda i: (i, 0)),
          pl.BlockSpec((D, D), lambda i: (0, 0)),
          pl.BlockSpec((D, 16), lambda i: (0, 0)),
          pl.BlockSpec((D, 16), lambda i: (0, 0)),
      ],
      out_specs=[
          pl.BlockSpec((_RB, D), lambda i: (i, 0)),
          pl.BlockSpec((_RB, D), lambda i: (i, 0)),
          pl.BlockSpec((_RB, D), lambda i: (i, 0)),
          pl.BlockSpec((_RB, 16), lambda i: (i, 0)),
          pl.BlockSpec((_RB, 16), lambda i: (i, 0)),
      ],
      out_shape=[
          jax.ShapeDtypeStruct((N, D), jnp.float32),
          jax.ShapeDtypeStruct((N, D), jnp.float32),
          jax.ShapeDtypeStruct((N, D), jnp.float32),
          jax.ShapeDtypeStruct((N, 16), jnp.float32),
          jax.ShapeDtypeStruct((N, 16), jnp.float32),
      ],
  )(x0, Z1, raw1, s1, W, wL, wR)


def _tc_final(x0, x1, Z2, Gacc, raw2, s2):
  def body(x0_ref, x1_ref, z_ref, gacc_ref, raw_ref, s_ref,
           e_ref, g_ref, reg_ref):
    i = pl.program_id(0)
    x0b = x0_ref[...]
    e_ref[...] = x0b + 2.0 * x1_ref[...] + z_ref[...]
    g_ref[...] = gacc_ref[...] + _gat_norm(raw_ref[...], s_ref[...])

    @pl.when(i == 0)
    def _():
      reg_ref[...] = jnp.zeros((1, 1), jnp.float32)

    reg_ref[...] += jnp.reshape(jnp.sum(x0b * x0b), (1, 1))

  grid = N // _RB
  return pl.pallas_call(
      body,
      grid=(grid,),
      in_specs=[
          pl.BlockSpec((_RB, D), lambda i: (i, 0)),
          pl.BlockSpec((_RB, D), lambda i: (i, 0)),
          pl.BlockSpec((_RB, D), lambda i: (i, 0)),
          pl.BlockSpec((_RB, D), lambda i: (i, 0)),
          pl.BlockSpec((_RB, D), lambda i: (i, 0)),
          pl.BlockSpec((_RB, 16), lambda i: (i, 0)),
      ],
      out_specs=[
          pl.BlockSpec((_RB, D), lambda i: (i, 0)),
          pl.BlockSpec((_RB, D), lambda i: (i, 0)),
          pl.BlockSpec((1, 1), lambda i: (0, 0)),
      ],
      out_shape=[
          jax.ShapeDtypeStruct((N, D), jnp.float32),
          jax.ShapeDtypeStruct((N, D), jnp.float32),
          jax.ShapeDtypeStruct((1, 1), jnp.float32),
      ],
  )(x0, x1, Z2, Gacc, raw2, s2)


_CB = 1000


def _tc_loss(batch, esum, regsum):
  nsteps = N_U // _CB

  def body(batch_ref, eu_ref, ei_ref, reg_ref,
           su_ref, si_ref, loss_ref, lr_ref, ls_ref):
    k = pl.program_id(0)

    @pl.when(k == 0)
    def _():
      su_ref[...] = jnp.zeros((B,), jnp.float32)
      si_ref[...] = jnp.zeros((B,), jnp.float32)

    gu = batch_ref[0:B, :]
    gi = batch_ref[B:2 * B, :]
    dn = (((1,), (1,)), ((), ()))
    lu = lax.dot_general(gu, eu_ref[...], dn,
                         preferred_element_type=jnp.float32)
    li = lax.dot_general(gi, ei_ref[...], dn,
                         preferred_element_type=jnp.float32)
    su_ref[...] += jnp.sum(jnp.exp(lu * (1.0 / TEMP)), axis=1)
    si_ref[...] += jnp.sum(jnp.exp(li * (1.0 / TEMP)), axis=1)

    @pl.when(k == nsteps - 1)
    def _():
      eu = batch_ref[2 * B:3 * B, :]
      ei = batch_ref[3 * B:4 * B, :]
      posb = batch_ref[4 * B:5 * B, :]
      negb = batch_ref[5 * B:6 * B, :]
      neg_score = (jnp.mean(jnp.log(su_ref[...] + 1e-8)) +
                   jnp.mean(jnp.log(si_ref[...] + 1e-8)))
      pos_score = (
          jnp.mean(jnp.log(jnp.exp(jnp.sum(gu * eu, axis=1) * (1.0 / TEMP)))) +
          jnp.mean(jnp.log(jnp.exp(jnp.sum(gi * ei, axis=1) * (1.0 / TEMP)))))
      loss_s = -pos_score + neg_score
      pos_sc = jnp.sum(eu * posb, axis=1)
      neg_sc = jnp.sum(eu * negb, axis=1)
      loss_r = -jnp.mean(jnp.log(jax.nn.sigmoid(pos_sc - neg_sc)))
      lr_ref[...] = jnp.reshape(loss_r, (1, 1))
      ls_ref[...] = jnp.reshape(LAMBDA_1 * loss_s, (1, 1))
      loss_ref[...] = (jnp.reshape(loss_r + LAMBDA_1 * loss_s, (1, 1)) +
                       reg_ref[...] * LAMBDA_2)

  return pl.pallas_call(
      body,
      grid=(nsteps,),
      in_specs=[
          pl.BlockSpec((6 * B, D), lambda k: (0, 0)),
          pl.BlockSpec((_CB, D), lambda k: (k, 0)),
          pl.BlockSpec((_CB, D), lambda k: (k + N_U // _CB, 0)),
          pl.BlockSpec((1, 1), lambda k: (0, 0)),
      ],
      out_specs=[
          pl.BlockSpec((B,), lambda k: (0,)),
          pl.BlockSpec((B,), lambda k: (0,)),
          pl.BlockSpec((1, 1), lambda k: (0, 0)),
          pl.BlockSpec((1, 1), lambda k: (0, 0)),
          pl.BlockSpec((1, 1), lambda k: (0, 0)),
      ],
      out_shape=[
          jax.ShapeDtypeStruct((B,), jnp.float32),
          jax.ShapeDtypeStruct((B,), jnp.float32),
          jax.ShapeDtypeStruct((1, 1), jnp.float32),
          jax.ShapeDtypeStruct((1, 1), jnp.float32),
          jax.ShapeDtypeStruct((1, 1), jnp.float32),
      ],
  )(batch, esum, esum, regsum)


def kernel(uids, iids, pos, neg, E_u_0, E_i_0, W_gat, attn_l, attn_r,
           adj_vals, edge_u, edge_i):
  i32 = jnp.int32
  uids = uids.astype(i32)
  iids = iids.astype(i32)
  pos = pos.astype(i32)
  neg = neg.astype(i32)
  edge_u = edge_u.astype(i32)
  edge_i = edge_i.astype(i32)
  adj_vals = adj_vals.astype(jnp.float32)

  x0 = jnp.concatenate([E_u_0, E_i_0], axis=0)
  W = W_gat.reshape(D, H * DH)
  wl = jnp.einsum("dhk,hk->dh", W_gat, attn_l)
  wr = jnp.einsum("dhk,hk->dh", W_gat, attn_r)
  wL = jnp.pad(wl, ((0, 0), (0, 16 - H)))
  wR = jnp.pad(wr, ((0, 0), (0, 16 - H)))

  padS = jnp.zeros((SP_LEN - E,), i32)
  padSf = jnp.zeros((SP_LEN - E,), jnp.float32)
  gidx = jnp.concatenate([edge_i + N_U, padS, edge_u, padS])
  sidx = jnp.concatenate([edge_u, padS, edge_i, padS])
  adj2 = jnp.concatenate([adj_vals, padSf, adj_vals, padSf])

  ar_u = jnp.arange(N_U, dtype=i32)
  npad = GA_LEN - (E + N_U)
  pad0 = jnp.zeros((npad,), i32)
  padD = jnp.full((npad,), N_U, i32)
  src = jnp.concatenate([edge_i + N_U, ar_u, pad0, edge_u, ar_u + N_U, pad0])
  dstg = jnp.concatenate([edge_u, ar_u, pad0, edge_i + N_U, ar_u + N_U, pad0])
  dstl = jnp.concatenate([edge_u, ar_u, padD, edge_i, ar_u, padD])

  idxg = jnp.concatenate([uids, iids + N_U])
  idxe = jnp.concatenate([uids, iids + N_U, pos + N_U, neg + N_U])

  h1, elT1, erT1 = _tc_pre(x0, W, wL, wR)
  Z1 = _sc_spmm(x0, gidx, sidx, adj2)
  raw1, s1 = _sc_gat(h1, elT1, erT1, src, dstg, dstl)
  raw1c = jnp.concatenate([raw1[:N_U], raw1[NLOC:NLOC + N_I]])
  s1c = jnp.concatenate([s1[:N_U], s1[NLOC:NLOC + N_I]])
  x1, Gacc, h2, elT2, erT2 = _tc_mid(x0, Z1, raw1c, s1c, W, wL, wR)

  Z2 = _sc_spmm(x1, gidx, sidx, adj2)
  raw2, s2 = _sc_gat(h2, elT2, erT2, src, dstg, dstl)
  raw2c = jnp.concatenate([raw2[:N_U], raw2[NLOC:NLOC + N_I]])
  s2c = jnp.concatenate([s2[:N_U], s2[NLOC:NLOC + N_I]])
  esum, gsum, regsum = _tc_final(x0, x1, Z2, Gacc, raw2c, s2c)

  batch = _sc_batch(gsum, esum, idxg, idxe)
  _, _, loss, lr, ls = _tc_loss(batch, esum, regsum)
  return (loss[0, 0], lr[0, 0], ls[0, 0])


# speedup vs baseline: 1.1424x; 1.0485x over previous
import functools
import jax
import jax.numpy as jnp
from jax import lax
from jax.experimental import pallas as pl
from jax.experimental.pallas import tpu as pltpu
from jax.experimental.pallas import tpu_sc as plsc

N_U = 10000
N_I = 10000
N = N_U + N_I
D = 128
E = 160000
H = 4
DH = 32
TEMP = 0.2
LAMBDA_1 = 0.2
LAMBDA_2 = 1e-07
NEG_SLOPE = 0.2
B = 1024

NS = 16
CH = 128

SP_PER_TILE = 10240
SP_CHUNKS = SP_PER_TILE // CH
SP_LEN = NS * SP_PER_TILE

GCH = 96
GA_PER_TILE = 10752
GA_CHUNKS = GA_PER_TILE // GCH
GA_LEN = NS * GA_PER_TILE
NLOC = 10016
STRIPE = NLOC // NS


def _mesh():
  return plsc.VectorSubcoreMesh(core_axis_name="c", subcore_axis_name="s")


_SC_PARAMS = pltpu.CompilerParams(use_tc_tiling_on_sc=False,
                                  needs_layout_passes=False)


def _splat(vec, lane):
  idx = jnp.full((16,), lane, jnp.int32)
  return jnp.take_along_axis(vec, idx, axis=0)


_PERM = []
for _j in range(4):
  for _k in range(16):
    _PERM.extend([32 * _j + _k, 32 * _j + 16 + _k])
_PERM = tuple(_PERM)


def _perm_cast(x):
  return jnp.take(x, jnp.array(_PERM, jnp.int32), axis=1).astype(jnp.bfloat16)


def _zero_vmem(ref, nrows, width):
  @pl.loop(0, nrows)
  def _(r):
    for j in range(width // 16):
      ref[r, pl.ds(16 * j, 16)] = jnp.zeros((16,), jnp.float32)


def _sc_spmm(x, gidx, sidx, adj):
  @functools.partial(
      pl.kernel,
      out_type=jax.ShapeDtypeStruct((2 * N_U, D), jnp.float32),
      mesh=_mesh(),
      compiler_params=_SC_PARAMS,
      scratch_types=[
          [pltpu.VMEM((CH,), jnp.int32) for _ in range(2)],
          [pltpu.VMEM((CH,), jnp.int32) for _ in range(2)],
          [pltpu.VMEM((CH,), jnp.float32) for _ in range(2)],
          [pltpu.VMEM((CH, D), jnp.bfloat16) for _ in range(2)],
          pltpu.VMEM((CH, D), jnp.float32),
          pltpu.VMEM_SHARED((N_U, D), jnp.float32),
          [pltpu.SemaphoreType.DMA for _ in range(2)],
          [pltpu.SemaphoreType.DMA for _ in range(2)],
      ],
  )
  def k(x_hbm, gidx_hbm, sidx_hbm, adj_hbm, out_hbm,
        gi, si, adj_v, rows, rows_f, acc, sem, isem):
    c = lax.axis_index("c")
    s = lax.axis_index("s")
    _zero_vmem(rows_f, CH, D)
    stripe = s * 625
    for q in range(4):
      pltpu.sync_copy(rows_f, acc.at[pl.ds(stripe + q * CH, CH)])
    pltpu.sync_copy(rows_f.at[pl.ds(0, 113)],
                    acc.at[pl.ds(stripe + 4 * CH, 113)])
    plsc.subcore_barrier()

    cbase = c * SP_LEN + s * SP_PER_TILE
    pltpu.sync_copy(gidx_hbm.at[pl.ds(cbase, CH)], gi[0])
    pltpu.sync_copy(sidx_hbm.at[pl.ds(cbase, CH)], si[0])
    pltpu.sync_copy(adj_hbm.at[pl.ds(cbase, CH)], adj_v[0])
    pltpu.async_copy(x_hbm.at[gi[0]], rows[0], sem[0])
    pltpu.async_copy(gidx_hbm.at[pl.ds(cbase + CH, CH)], gi[1], isem[1])
    pltpu.async_copy(sidx_hbm.at[pl.ds(cbase + CH, CH)], si[1], isem[1])
    pltpu.async_copy(adj_hbm.at[pl.ds(cbase + CH, CH)], adj_v[1], isem[1])

    @pl.loop(0, SP_CHUNKS // 2)
    def _(g2):
      for b in range(2):
        nb = 1 - b
        ci = 2 * g2 + b

        @pl.when(ci + 1 < SP_CHUNKS)
        def _():
          pltpu.make_async_copy(gidx_hbm.at[pl.ds(0, CH)], gi[nb],
                                isem[nb]).wait()
          pltpu.make_async_copy(sidx_hbm.at[pl.ds(0, CH)], si[nb],
                                isem[nb]).wait()
          pltpu.make_async_copy(adj_hbm.at[pl.ds(0, CH)], adj_v[nb],
                                isem[nb]).wait()
          pltpu.async_copy(x_hbm.at[gi[nb]], rows[nb], sem[nb])

        pltpu.make_async_copy(x_hbm.at[pl.ds(0, CH)], rows[b], sem[b]).wait()

        @pl.loop(0, CH // 16)
        def _(q):
          av16 = adj_v[b][pl.ds(16 * q, 16)]
          for r in range(16):
            av = _splat(av16, r)
            rr = 16 * q + r
            for j in range(D // 32):
              v32 = rows[b][rr, pl.ds(32 * j, 32)]
              lo, hi = plsc.unpack(v32, format=plsc.PackFormat.INTERLEAVED)
              rows_f[rr, pl.ds(32 * j, 16)] = lo * av
              rows_f[rr, pl.ds(32 * j + 16, 16)] = hi * av

        pltpu.sync_copy(rows_f, acc.at[si[b]], add=True)

        @pl.when(ci + 2 < SP_CHUNKS)
        def _():
          b2 = cbase + (ci + 2) * CH
          pltpu.async_copy(gidx_hbm.at[pl.ds(b2, CH)], gi[b], isem[b])
          pltpu.async_copy(sidx_hbm.at[pl.ds(b2, CH)], si[b], isem[b])
          pltpu.async_copy(adj_hbm.at[pl.ds(b2, CH)], adj_v[b], isem[b])

    plsc.subcore_barrier()
    for q in range(4):
      pltpu.sync_copy(acc.at[pl.ds(stripe + q * CH, CH)],
                      out_hbm.at[pl.ds(c * N_U + stripe + q * CH, CH)])
    pltpu.sync_copy(acc.at[pl.ds(stripe + 4 * CH, 113)],
                    out_hbm.at[pl.ds(c * N_U + stripe + 4 * CH, 113)])

  return k(x, gidx, sidx, adj)


def _sc_gat(h, elT, erT, src, dstg, dstl):
  @functools.partial(
      pl.kernel,
      out_type=(jax.ShapeDtypeStruct((2 * NLOC, D), jnp.float32),
                jax.ShapeDtypeStruct((2 * NLOC, 16), jnp.float32)),
      mesh=_mesh(),
      compiler_params=_SC_PARAMS,
      scratch_types=[
          [pltpu.VMEM((GCH,), jnp.int32) for _ in range(2)],
          [pltpu.VMEM((GCH,), jnp.int32) for _ in range(2)],
          [pltpu.VMEM((GCH,), jnp.int32) for _ in range(2)],
          [pltpu.VMEM((GCH, 16), jnp.float32) for _ in range(2)],
          [pltpu.VMEM((GCH, 16), jnp.float32) for _ in range(2)],
          [pltpu.VMEM((GCH, D), jnp.float32) for _ in range(2)],
          [pltpu.VMEM((GCH, 16), jnp.float32) for _ in range(2)],
          pltpu.VMEM_SHARED((NLOC, D), jnp.float32),
          pltpu.VMEM_SHARED((NLOC, 16), jnp.float32),
          [pltpu.SemaphoreType.DMA for _ in range(2)],
          [pltpu.SemaphoreType.DMA for _ in range(2)],
      ],
  )
  def k(h_hbm, elT_hbm, erT_hbm, src_hbm, dstg_hbm, dstl_hbm, raw_hbm, s_hbm,
        si, dg, di, el_s, er_d, hrows, trows, acc, sacc, sem, isem):
    c = lax.axis_index("c")
    s = lax.axis_index("s")
    _zero_vmem(hrows[0], GCH, D)
    _zero_vmem(trows[0], GCH, 16)
    stripe = s * STRIPE
    nfull, rem = STRIPE // GCH, STRIPE % GCH
    for q in range(nfull):
      pltpu.sync_copy(hrows[0], acc.at[pl.ds(stripe + q * GCH, GCH)])
      pltpu.sync_copy(trows[0], sacc.at[pl.ds(stripe + q * GCH, GCH)])
    if rem:
      pltpu.sync_copy(hrows[0].at[pl.ds(0, rem)],
                      acc.at[pl.ds(stripe + nfull * GCH, rem)])
      pltpu.sync_copy(trows[0].at[pl.ds(0, rem)],
                      sacc.at[pl.ds(stripe + nfull * GCH, rem)])
    plsc.subcore_barrier()

    cbase = c * GA_LEN + s * GA_PER_TILE
    pltpu.sync_copy(src_hbm.at[pl.ds(cbase, GCH)], si[0])
    pltpu.sync_copy(dstg_hbm.at[pl.ds(cbase, GCH)], dg[0])
    pltpu.sync_copy(dstl_hbm.at[pl.ds(cbase, GCH)], di[0])
    pltpu.async_copy(elT_hbm.at[si[0]], el_s[0], sem[0])
    pltpu.async_copy(erT_hbm.at[dg[0]], er_d[0], sem[0])
    pltpu.async_copy(h_hbm.at[si[0]], hrows[0], sem[0])
    pltpu.async_copy(src_hbm.at[pl.ds(cbase + GCH, GCH)], si[1], isem[1])
    pltpu.async_copy(dstg_hbm.at[pl.ds(cbase + GCH, GCH)], dg[1], isem[1])
    pltpu.async_copy(dstl_hbm.at[pl.ds(cbase + GCH, GCH)], di[1], isem[1])

    @pl.loop(0, GA_CHUNKS // 2)
    def _(g2):
      for b in range(2):
        nb = 1 - b
        ci = 2 * g2 + b

        @pl.when(ci + 1 < GA_CHUNKS)
        def _():
          pltpu.make_async_copy(src_hbm.at[pl.ds(0, GCH)], si[nb],
                                isem[nb]).wait()
          pltpu.make_async_copy(src_hbm.at[pl.ds(0, GCH)], dg[nb],
                                isem[nb]).wait()
          pltpu.make_async_copy(src_hbm.at[pl.ds(0, GCH)], di[nb],
                                isem[nb]).wait()
          pltpu.async_copy(elT_hbm.at[si[nb]], el_s[nb], sem[nb])
          pltpu.async_copy(erT_hbm.at[dg[nb]], er_d[nb], sem[nb])
          pltpu.async_copy(h_hbm.at[si[nb]], hrows[nb], sem[nb])

        pltpu.make_async_copy(elT_hbm.at[pl.ds(0, GCH)], el_s[b],
                              sem[b]).wait()
        pltpu.make_async_copy(elT_hbm.at[pl.ds(0, GCH)], er_d[b],
                              sem[b]).wait()
        pltpu.make_async_copy(h_hbm.at[pl.ds(0, GCH)], hrows[b],
                              sem[b]).wait()

        @pl.loop(0, GCH)
        def _(r):
          v = el_s[b][r, :] + er_d[b][r, :]
          t = jnp.exp(jnp.maximum(v, v * NEG_SLOPE))
          trows[b][r, :] = t
          for hh in range(H):
            tv = _splat(t, hh)
            hrows[b][r, pl.ds(DH * hh, 16)] = (
                hrows[b][r, pl.ds(DH * hh, 16)] * tv)
            hrows[b][r, pl.ds(DH * hh + 16, 16)] = (
                hrows[b][r, pl.ds(DH * hh + 16, 16)] * tv)

        pltpu.sync_copy(trows[b], sacc.at[di[b]], add=True)
        pltpu.sync_copy(hrows[b], acc.at[di[b]], add=True)

        @pl.when(ci + 2 < GA_CHUNKS)
        def _():
          b2 = cbase + (ci + 2) * GCH
          pltpu.async_copy(src_hbm.at[pl.ds(b2, GCH)], si[b], isem[b])
          pltpu.async_copy(dstg_hbm.at[pl.ds(b2, GCH)], dg[b], isem[b])
          pltpu.async_copy(dstl_hbm.at[pl.ds(b2, GCH)], di[b], isem[b])

    plsc.subcore_barrier()
    for q in range(nfull):
      pltpu.sync_copy(acc.at[pl.ds(stripe + q * GCH, GCH)],
                      raw_hbm.at[pl.ds(c * NLOC + stripe + q * GCH, GCH)])
      pltpu.sync_copy(sacc.at[pl.ds(stripe + q * GCH, GCH)],
                      s_hbm.at[pl.ds(c * NLOC + stripe + q * GCH, GCH)])
    if rem:
      off = stripe + nfull * GCH
      pltpu.sync_copy(acc.at[pl.ds(off, rem)],
                      raw_hbm.at[pl.ds(c * NLOC + off, rem)])
      pltpu.sync_copy(sacc.at[pl.ds(off, rem)],
                      s_hbm.at[pl.ds(c * NLOC + off, rem)])

  return k(h, elT, erT, src, dstg, dstl)


def _sc_batch(gsum, esum, idxg, idxe):
  @functools.partial(
      pl.kernel,
      out_type=jax.ShapeDtypeStruct((6 * B, D), jnp.float32),
      mesh=_mesh(),
      compiler_params=_SC_PARAMS,
      scratch_types=[
          pltpu.VMEM((CH,), jnp.int32),
          pltpu.VMEM((CH, D), jnp.float32),
          pltpu.SemaphoreType.DMA,
      ],
  )
  def k(g_hbm, e_hbm, idxg_hbm, idxe_hbm, out_hbm, idx_v, rows, sem):
    c = lax.axis_index("c")
    s = lax.axis_index("s")

    @pl.when(c == 0)
    def _():
      pltpu.sync_copy(idxg_hbm.at[pl.ds(s * CH, CH)], idx_v)
      pltpu.async_copy(g_hbm.at[idx_v], rows, sem).wait()
      pltpu.sync_copy(rows, out_hbm.at[pl.ds(s * CH, CH)])

    @pl.when(c == 1)
    def _():
      for q in range(2):
        pltpu.sync_copy(idxe_hbm.at[pl.ds(s * 2 * CH + q * CH, CH)], idx_v)
        pltpu.async_copy(e_hbm.at[idx_v], rows, sem).wait()
        pltpu.sync_copy(rows, out_hbm.at[pl.ds(2 * B + s * 2 * CH + q * CH, CH)])

  return k(gsum, esum, idxg, idxe)


_RB = 2000


def _tc_pre(x, W, wL, wR):
  def body(x_ref, w_ref, wl_ref, wr_ref, h_ref, el_ref, er_ref):
    xb = x_ref[...]
    h_ref[...] = jnp.dot(xb, w_ref[...], preferred_element_type=jnp.float32)
    el_ref[...] = jnp.dot(xb, wl_ref[...], preferred_element_type=jnp.float32)
    er_ref[...] = jnp.dot(xb, wr_ref[...], preferred_element_type=jnp.float32)

  grid = N // _RB
  return pl.pallas_call(
      body,
      grid=(grid,),
      in_specs=[
          pl.BlockSpec((_RB, D), lambda i: (i, 0)),
          pl.BlockSpec((D, D), lambda i: (0, 0)),
          pl.BlockSpec((D, 16), lambda i: (0, 0)),
          pl.BlockSpec((D, 16), lambda i: (0, 0)),
      ],
      out_specs=[
          pl.BlockSpec((_RB, D), lambda i: (i, 0)),
          pl.BlockSpec((_RB, 16), lambda i: (i, 0)),
          pl.BlockSpec((_RB, 16), lambda i: (i, 0)),
      ],
      out_shape=[
          jax.ShapeDtypeStruct((N, D), jnp.float32),
          jax.ShapeDtypeStruct((N, 16), jnp.float32),
          jax.ShapeDtypeStruct((N, 16), jnp.float32),
      ],
  )(x, W, wL, wR)


def _gat_norm(raw, sden):
  n = raw.shape[0]
  den = sden[:, :H] + 1e-9
  g3 = raw.reshape(n, H, DH) / den[:, :, None]
  g = g3.reshape(n, D)
  return jnp.where(g > 0, g, jnp.exp(g) - 1.0)


def _tc_mid(x0, Z1, raw1, s1, W, wL, wR):
  def body(x0_ref, z_ref, raw_ref, s_ref, w_ref, wl_ref, wr_ref,
           x1_ref, gacc_ref, h_ref, el_ref, er_ref):
    x0b = x0_ref[...]
    x1b = x0b + z_ref[...]
    x1_ref[...] = x1b
    gacc_ref[...] = x0b + _gat_norm(raw_ref[...], s_ref[...])
    h_ref[...] = jnp.dot(x1b, w_ref[...], preferred_element_type=jnp.float32)
    el_ref[...] = jnp.dot(x1b, wl_ref[...], preferred_element_type=jnp.float32)
    er_ref[...] = jnp.dot(x1b, wr_ref[...], preferred_element_type=jnp.float32)

  grid = N // _RB
  return pl.pallas_call(
      body,
      grid=(grid,),
      in_specs=[
          pl.BlockSpec((_RB, D), lambda i: (i, 0)),
          pl.BlockSpec((_RB, D), lambda i: (i, 0)),
          pl.BlockSpec((_RB, D), lambda i: (i, 0)),
          pl.BlockSpec((_RB, 16), lambda i: (i, 0)),
          pl.BlockSpec((D, D), lambda i: (0, 0)),
          pl.BlockSpec((D, 16), lambda i: (0, 0)),
          pl.BlockSpec((D, 16), lambda i: (0, 0)),
      ],
      out_specs=[
          pl.BlockSpec((_RB, D), lambda i: (i, 0)),
          pl.BlockSpec((_RB, D), lambda i: (i, 0)),
          pl.BlockSpec((_RB, D), lambda i: (i, 0)),
          pl.BlockSpec((_RB, 16), lambda i: (i, 0)),
          pl.BlockSpec((_RB, 16), lambda i: (i, 0)),
      ],
      out_shape=[
          jax.ShapeDtypeStruct((N, D), jnp.float32),
          jax.ShapeDtypeStruct((N, D), jnp.float32),
          jax.ShapeDtypeStruct((N, D), jnp.float32),
          jax.ShapeDtypeStruct((N, 16), jnp.float32),
          jax.ShapeDtypeStruct((N, 16), jnp.float32),
      ],
  )(x0, Z1, raw1, s1, W, wL, wR)


def _tc_final(x0, x1, Z2, Gacc, raw2, s2):
  def body(x0_ref, x1_ref, z_ref, gacc_ref, raw_ref, s_ref,
           e_ref, g_ref, reg_ref):
    i = pl.program_id(0)
    x0b = x0_ref[...]
    e_ref[...] = x0b + 2.0 * x1_ref[...] + z_ref[...]
    g_ref[...] = gacc_ref[...] + _gat_norm(raw_ref[...], s_ref[...])

    @pl.when(i == 0)
    def _():
      reg_ref[...] = jnp.zeros((1, 1), jnp.float32)

    reg_ref[...] += jnp.reshape(jnp.sum(x0b * x0b), (1, 1))

  grid = N // _RB
  return pl.pallas_call(
      body,
      grid=(grid,),
      in_specs=[
          pl.BlockSpec((_RB, D), lambda i: (i, 0)),
          pl.BlockSpec((_RB, D), lambda i: (i, 0)),
          pl.BlockSpec((_RB, D), lambda i: (i, 0)),
          pl.BlockSpec((_RB, D), lambda i: (i, 0)),
          pl.BlockSpec((_RB, D), lambda i: (i, 0)),
          pl.BlockSpec((_RB, 16), lambda i: (i, 0)),
      ],
      out_specs=[
          pl.BlockSpec((_RB, D), lambda i: (i, 0)),
          pl.BlockSpec((_RB, D), lambda i: (i, 0)),
          pl.BlockSpec((1, 1), lambda i: (0, 0)),
      ],
      out_shape=[
          jax.ShapeDtypeStruct((N, D), jnp.float32),
          jax.ShapeDtypeStruct((N, D), jnp.float32),
          jax.ShapeDtypeStruct((1, 1), jnp.float32),
      ],
  )(x0, x1, Z2, Gacc, raw2, s2)


_CB = 1000


def _tc_loss(batch, esum, regsum):
  nsteps = N_U // _CB

  def body(batch_ref, eu_ref, ei_ref, reg_ref,
           su_ref, si_ref, loss_ref, lr_ref, ls_ref):
    k = pl.program_id(0)

    @pl.when(k == 0)
    def _():
      su_ref[...] = jnp.zeros((B,), jnp.float32)
      si_ref[...] = jnp.zeros((B,), jnp.float32)

    gu = batch_ref[0:B, :]
    gi = batch_ref[B:2 * B, :]
    dn = (((1,), (1,)), ((), ()))
    lu = lax.dot_general(gu, eu_ref[...], dn,
                         preferred_element_type=jnp.float32)
    li = lax.dot_general(gi, ei_ref[...], dn,
                         preferred_element_type=jnp.float32)
    su_ref[...] += jnp.sum(jnp.exp(lu * (1.0 / TEMP)), axis=1)
    si_ref[...] += jnp.sum(jnp.exp(li * (1.0 / TEMP)), axis=1)

    @pl.when(k == nsteps - 1)
    def _():
      eu = batch_ref[2 * B:3 * B, :]
      ei = batch_ref[3 * B:4 * B, :]
      posb = batch_ref[4 * B:5 * B, :]
      negb = batch_ref[5 * B:6 * B, :]
      neg_score = (jnp.mean(jnp.log(su_ref[...] + 1e-8)) +
                   jnp.mean(jnp.log(si_ref[...] + 1e-8)))
      pos_score = (
          jnp.mean(jnp.log(jnp.exp(jnp.sum(gu * eu, axis=1) * (1.0 / TEMP)))) +
          jnp.mean(jnp.log(jnp.exp(jnp.sum(gi * ei, axis=1) * (1.0 / TEMP)))))
      loss_s = -pos_score + neg_score
      pos_sc = jnp.sum(eu * posb, axis=1)
      neg_sc = jnp.sum(eu * negb, axis=1)
      loss_r = -jnp.mean(jnp.log(jax.nn.sigmoid(pos_sc - neg_sc)))
      lr_ref[...] = jnp.reshape(loss_r, (1, 1))
      ls_ref[...] = jnp.reshape(LAMBDA_1 * loss_s, (1, 1))
      loss_ref[...] = (jnp.reshape(loss_r + LAMBDA_1 * loss_s, (1, 1)) +
                       reg_ref[...] * LAMBDA_2)

  return pl.pallas_call(
      body,
      grid=(nsteps,),
      in_specs=[
          pl.BlockSpec((6 * B, D), lambda k: (0, 0)),
          pl.BlockSpec((_CB, D), lambda k: (k, 0)),
          pl.BlockSpec((_CB, D), lambda k: (k + N_U // _CB, 0)),
          pl.BlockSpec((1, 1), lambda k: (0, 0)),
      ],
      out_specs=[
          pl.BlockSpec((B,), lambda k: (0,)),
          pl.BlockSpec((B,), lambda k: (0,)),
          pl.BlockSpec((1, 1), lambda k: (0, 0)),
          pl.BlockSpec((1, 1), lambda k: (0, 0)),
          pl.BlockSpec((1, 1), lambda k: (0, 0)),
      ],
      out_shape=[
          jax.ShapeDtypeStruct((B,), jnp.float32),
          jax.ShapeDtypeStruct((B,), jnp.float32),
          jax.ShapeDtypeStruct((1, 1), jnp.float32),
          jax.ShapeDtypeStruct((1, 1), jnp.float32),
          jax.ShapeDtypeStruct((1, 1), jnp.float32),
      ],
  )(batch, esum, esum, regsum)


def kernel(uids, iids, pos, neg, E_u_0, E_i_0, W_gat, attn_l, attn_r,
           adj_vals, edge_u, edge_i):
  i32 = jnp.int32
  uids = uids.astype(i32)
  iids = iids.astype(i32)
  pos = pos.astype(i32)
  neg = neg.astype(i32)
  edge_u = edge_u.astype(i32)
  edge_i = edge_i.astype(i32)
  adj_vals = adj_vals.astype(jnp.float32)

  x0 = jnp.concatenate([E_u_0, E_i_0], axis=0)
  W = W_gat.reshape(D, H * DH)
  wl = jnp.einsum("dhk,hk->dh", W_gat, attn_l)
  wr = jnp.einsum("dhk,hk->dh", W_gat, attn_r)
  wL = jnp.pad(wl, ((0, 0), (0, 16 - H)))
  wR = jnp.pad(wr, ((0, 0), (0, 16 - H)))

  padS = jnp.zeros((SP_LEN - E,), i32)
  padSf = jnp.zeros((SP_LEN - E,), jnp.float32)
  gidx = jnp.concatenate([edge_i + N_U, padS, edge_u, padS])
  sidx = jnp.concatenate([edge_u, padS, edge_i, padS])
  adj2 = jnp.concatenate([adj_vals, padSf, adj_vals, padSf])

  ar_u = jnp.arange(N_U, dtype=i32)
  npad = GA_LEN - (E + N_U)
  pad0 = jnp.zeros((npad,), i32)
  padD = jnp.full((npad,), N_U, i32)
  src = jnp.concatenate([edge_i + N_U, ar_u, pad0, edge_u, ar_u + N_U, pad0])
  dstg = jnp.concatenate([edge_u, ar_u, pad0, edge_i + N_U, ar_u + N_U, pad0])
  dstl = jnp.concatenate([edge_u, ar_u, padD, edge_i, ar_u, padD])

  idxg = jnp.concatenate([uids, iids + N_U])
  idxe = jnp.concatenate([uids, iids + N_U, pos + N_U, neg + N_U])

  h1, elT1, erT1 = _tc_pre(x0, W, wL, wR)
  Z1 = _sc_spmm(_perm_cast(x0), gidx, sidx, adj2)
  raw1, s1 = _sc_gat(h1, elT1, erT1, src, dstg, dstl)
  raw1c = jnp.concatenate([raw1[:N_U], raw1[NLOC:NLOC + N_I]])
  s1c = jnp.concatenate([s1[:N_U], s1[NLOC:NLOC + N_I]])
  x1, Gacc, h2, elT2, erT2 = _tc_mid(x0, Z1, raw1c, s1c, W, wL, wR)

  Z2 = _sc_spmm(_perm_cast(x1), gidx, sidx, adj2)
  raw2, s2 = _sc_gat(h2, elT2, erT2, src, dstg, dstl)
  raw2c = jnp.concatenate([raw2[:N_U], raw2[NLOC:NLOC + N_I]])
  s2c = jnp.concatenate([s2[:N_U], s2[NLOC:NLOC + N_I]])
  esum, gsum, regsum = _tc_final(x0, x1, Z2, Gacc, raw2c, s2c)

  batch = _sc_batch(gsum, esum, idxg, idxe)
  _, _, loss, lr, ls = _tc_loss(batch, esum, regsum)
  return (loss[0, 0], lr[0, 0], ls[0, 0])
